# baseline (speedup 1.0000x reference)
import functools

import jax
import jax.numpy as jnp
from jax import lax
from jax.experimental import pallas as pl
from jax.experimental.pallas import tpu as pltpu
from jax.experimental.pallas import tpu_sc as plsc

BATCH = 16384
DIM = 32
N_TABLES = 6
NC = 2
NS = 16
NW = NC * NS
CHUNK = 128
ROWS_PER_W = BATCH // NW
CHUNKS_PER_W = ROWS_PER_W // CHUNK


def _sc_gather_body(idx_hbm, t0, t1, t2, t3, t4, t5,
                    o0, o1, o2, o3, o4, o5,
                    idx_v, rows_v, sem):
  tables = (t0, t1, t2, t3, t4, t5)
  outs = (o0, o1, o2, o3, o4, o5)
  wid = lax.axis_index("c") * NS + lax.axis_index("s")
  cbase = wid * CHUNKS_PER_W
  pltpu.sync_copy(idx_hbm.at[:, pl.ds(cbase, CHUNKS_PER_W), :], idx_v)
  copies = []
  for t in range(N_TABLES):
    for j in range(CHUNKS_PER_W):
      copies.append(pltpu.async_copy(
          tables[t].at[idx_v.at[t, j]], rows_v.at[t, j], sem))
  for c in copies:
    c.wait()
  for t in range(N_TABLES):
    pltpu.sync_copy(rows_v.at[t], outs[t].at[pl.ds(cbase, CHUNKS_PER_W)])


def _sc_gather(idx_all, user_emb, item_emb, genre_emb, lang_emb, age_emb,
               gender_emb):
  mesh = plsc.VectorSubcoreMesh(core_axis_name="c", subcore_axis_name="s",
                                num_cores=NC, num_subcores=NS)
  out_t = [jax.ShapeDtypeStruct((BATCH // CHUNK, CHUNK, DIM), jnp.float32)
           for _ in range(N_TABLES)]
  k = pl.kernel(
      _sc_gather_body,
      out_type=out_t,
      mesh=mesh,
      scratch_types=[
          pltpu.VMEM((N_TABLES, CHUNKS_PER_W, CHUNK), jnp.int32),
          pltpu.VMEM((N_TABLES, CHUNKS_PER_W, CHUNK, DIM), jnp.float32),
          pltpu.SemaphoreType.DMA,
      ],
      name="ncf_sc_gather",
      compiler_params=pltpu.CompilerParams(use_tc_tiling_on_sc=False),
  )
  return k(idx_all, user_emb, item_emb, genre_emb, lang_emb, age_emb,
           gender_emb)


def _mlp_body(x0, x1, x2, x3, x4, x5,
              w10, w11, w12, w13, w14, w15,
              b1, w2t, b2, w3t, b3, out):
  xs = (x0, x1, x2, x3, x4, x5)
  ws = (w10, w11, w12, w13, w14, w15)
  acc = jnp.zeros((x0.shape[0], 64), jnp.float32) + b1[...]
  for t in range(N_TABLES):
    acc = acc + jnp.dot(xs[t][...], ws[t][...],
                        preferred_element_type=jnp.float32)
  h1 = jnp.maximum(acc, 0.0)
  h2 = jnp.maximum(jnp.dot(h1, w2t[...], preferred_element_type=jnp.float32)
                   + b2[...], 0.0)
  z = jnp.dot(h2, w3t[...], preferred_element_type=jnp.float32) + b3[...]
  out[...] = 1.0 / (1.0 + jnp.exp(-z))


def _mlp(xs, w1ts, b1, w2t, b2, w3t, b3):
  bm = 2048
  grid = (BATCH // bm,)
  x_spec = pl.BlockSpec((bm, DIM), lambda i: (i, 0))
  full = lambda shape: pl.BlockSpec(shape, lambda i: (0, 0))
  in_specs = ([x_spec] * N_TABLES
              + [full((DIM, 64))] * N_TABLES
              + [full((1, 64)), full((64, 32)), full((1, 32)),
                 full((32, 1)), full((1, 1))])
  return pl.pallas_call(
      _mlp_body,
      grid=grid,
      in_specs=in_specs,
      out_specs=pl.BlockSpec((bm, 1), lambda i: (i, 0)),
      out_shape=jax.ShapeDtypeStruct((BATCH, 1), jnp.float32),
  )(*xs, *w1ts, b1, w2t, b2, w3t, b3)


def kernel(user_id, item_id, genre_id, language_id, age, gender,
           user_emb, item_emb, genre_emb, lang_emb, age_emb, gender_emb,
           W1, b1, W2, b2, W3, b3):
  idx_all = jnp.stack([
      user_id.astype(jnp.int32), item_id.astype(jnp.int32),
      genre_id.astype(jnp.int32), language_id.astype(jnp.int32),
      age.astype(jnp.int32), gender.astype(jnp.int32),
  ]).reshape(N_TABLES, BATCH // CHUNK, CHUNK)

  gathered = _sc_gather(idx_all, user_emb, item_emb, genre_emb, lang_emb,
                        age_emb, gender_emb)
  xs = [g.reshape(BATCH, DIM) for g in gathered]

  w1t = W1.T
  w1ts = [w1t[DIM * t:DIM * (t + 1)] for t in range(N_TABLES)]
  out = _mlp(xs, w1ts, b1.reshape(1, 64), W2.T, b2.reshape(1, 32),
             W3.T, b3.reshape(1, 1))
  return out


# speedup vs baseline: 1.1895x; 1.1895x over previous
import jax
import jax.numpy as jnp
from jax import lax
from jax.experimental import pallas as pl
from jax.experimental.pallas import tpu as pltpu
from jax.experimental.pallas import tpu_sc as plsc

BATCH = 16384
DIM = 32
NC = 2
NS = 16
NW = NC * NS
CHUNK = 128
ROWS_PER_W = BATCH // NW
CHUNKS_PER_W = ROWS_PER_W // CHUNK


def _sc_gather_body(idx_hbm, t0, t1, o0, o1, idx_v, rows_v, sem):
  tables = (t0, t1)
  outs = (o0, o1)
  wid = lax.axis_index("c") * NS + lax.axis_index("s")
  cbase = wid * CHUNKS_PER_W
  pltpu.sync_copy(idx_hbm.at[:, pl.ds(cbase, CHUNKS_PER_W), :], idx_v)
  copies = []
  for t in range(2):
    for j in range(CHUNKS_PER_W):
      copies.append(pltpu.async_copy(
          tables[t].at[idx_v.at[t, j]], rows_v.at[t, j], sem))
  for c in copies:
    c.wait()
  for t in range(2):
    pltpu.sync_copy(rows_v.at[t], outs[t].at[pl.ds(cbase, CHUNKS_PER_W)])


def _sc_gather(idx_all, user_emb, item_emb):
  mesh = plsc.VectorSubcoreMesh(core_axis_name="c", subcore_axis_name="s",
                                num_cores=NC, num_subcores=NS)
  out_t = [jax.ShapeDtypeStruct((BATCH // CHUNK, CHUNK, DIM), jnp.float32)
           for _ in range(2)]
  k = pl.kernel(
      _sc_gather_body,
      out_type=out_t,
      mesh=mesh,
      scratch_types=[
          pltpu.VMEM((2, CHUNKS_PER_W, CHUNK), jnp.int32),
          pltpu.VMEM((2, CHUNKS_PER_W, CHUNK, DIM), jnp.float32),
          pltpu.SemaphoreType.DMA,
      ],
      name="ncf_sc_gather",
      compiler_params=pltpu.CompilerParams(use_tc_tiling_on_sc=False),
  )
  return k(idx_all, user_emb, item_emb)


_SMALL_SIZES = (50, 20, 100, 2)


def _mlp_body(xu, xi, gid, lid, aid, gnd,
              genre_t, lang_t, age_t, gender_t,
              w1u, w1i, w1g, w1l, w1a, w1n,
              b1, w2t, b2, w3t, b3, out):
  bm = xu.shape[0]
  acc = jnp.dot(xu[...], w1u[...], preferred_element_type=jnp.float32)
  acc = acc + jnp.dot(xi[...], w1i[...], preferred_element_type=jnp.float32)
  ids = (gid, lid, aid, gnd)
  tabs = (genre_t, lang_t, age_t, gender_t)
  ws = (w1g, w1l, w1a, w1n)
  for t in range(4):
    ncat = _SMALL_SIZES[t]
    proj = jnp.dot(tabs[t][...], ws[t][...],
                   preferred_element_type=jnp.float32)
    cats = jax.lax.broadcasted_iota(jnp.int32, (bm, ncat), 1)
    onehot = (ids[t][...] == cats).astype(jnp.float32)
    acc = acc + jnp.dot(onehot, proj, preferred_element_type=jnp.float32)
  h1 = jnp.maximum(acc + b1[...], 0.0)
  h2 = jnp.maximum(jnp.dot(h1, w2t[...], preferred_element_type=jnp.float32)
                   + b2[...], 0.0)
  z = jnp.dot(h2, w3t[...], preferred_element_type=jnp.float32) + b3[...]
  out[...] = 1.0 / (1.0 + jnp.exp(-z))


def _mlp(xu, xi, gid, lid, aid, gnd, genre_emb, lang_emb, age_emb, gender_emb,
         w1ts, b1, w2t, b2, w3t, b3):
  bm = 2048
  grid = (BATCH // bm,)
  x_spec = pl.BlockSpec((bm, DIM), lambda i: (i, 0))
  id_spec = pl.BlockSpec((bm, 1), lambda i: (i, 0))
  full = lambda shape: pl.BlockSpec(shape, lambda i: (0, 0))
  in_specs = ([x_spec, x_spec] + [id_spec] * 4
              + [full((n, DIM)) for n in _SMALL_SIZES]
              + [full((DIM, 64))] * 6
              + [full((1, 64)), full((64, 32)), full((1, 32)),
                 full((32, 1)), full((1, 1))])
  return pl.pallas_call(
      _mlp_body,
      grid=grid,
      in_specs=in_specs,
      out_specs=pl.BlockSpec((bm, 1), lambda i: (i, 0)),
      out_shape=jax.ShapeDtypeStruct((BATCH, 1), jnp.float32),
  )(xu, xi, gid.reshape(BATCH, 1), lid.reshape(BATCH, 1),
    aid.reshape(BATCH, 1), gnd.reshape(BATCH, 1),
    genre_emb, lang_emb, age_emb, gender_emb, *w1ts, b1, w2t, b2, w3t, b3)


def kernel(user_id, item_id, genre_id, language_id, age, gender,
           user_emb, item_emb, genre_emb, lang_emb, age_emb, gender_emb,
           W1, b1, W2, b2, W3, b3):
  idx_all = jnp.stack([
      user_id.astype(jnp.int32), item_id.astype(jnp.int32),
  ]).reshape(2, BATCH // CHUNK, CHUNK)

  xu3, xi3 = _sc_gather(idx_all, user_emb, item_emb)
  xu = xu3.reshape(BATCH, DIM)
  xi = xi3.reshape(BATCH, DIM)

  w1t = W1.T
  w1ts = [w1t[DIM * t:DIM * (t + 1)] for t in range(6)]
  return _mlp(xu, xi, genre_id.astype(jnp.int32), language_id.astype(jnp.int32),
              age.astype(jnp.int32), gender.astype(jnp.int32),
              genre_emb, lang_emb, age_emb, gender_emb,
              w1ts, b1.reshape(1, 64), W2.T, b2.reshape(1, 32),
              W3.T, b3.reshape(1, 1))
